# baseline (speedup 1.0000x reference)
import functools

import jax
import jax.numpy as jnp
import numpy as np
from jax import lax
from jax.experimental import pallas as pl
from jax.experimental.pallas import tpu as pltpu
from jax.experimental.pallas import tpu_sc as plsc

V = 1000
D = 128
B = 1024
ND = 204800
NQ = 20480
H = 100
RATE = 0.1
INV_KEEP = 1.0 / (1.0 - RATE)

def _threefry2x32(ks0, ks1, x0, x1):
    u32 = np.uint32
    ks2 = u32(ks0 ^ ks1 ^ u32(0x1BD11BDA))
    keys = (u32(ks0), u32(ks1), ks2)
    x0 = (x0 + keys[0]).astype(u32)
    x1 = (x1 + keys[1]).astype(u32)
    rot_a = (13, 15, 26, 6)
    rot_b = (17, 29, 16, 24)
    for i, rots in enumerate((rot_a, rot_b, rot_a, rot_b, rot_a)):
        for r in rots:
            x0 = (x0 + x1).astype(u32)
            x1 = ((x1 << u32(r)) | (x1 >> u32(32 - r))).astype(u32)
            x1 = (x1 ^ x0).astype(u32)
        x0 = (x0 + keys[(i + 1) % 3]).astype(u32)
        x1 = (x1 + keys[(i + 2) % 3] + u32(i + 1)).astype(u32)
    return x0, x1


def _np_bernoulli(key, p, n):
    o0, o1 = _threefry2x32(key[0], key[1], np.zeros(n, np.uint32),
                           np.arange(n, dtype=np.uint32))
    bits = o0 ^ o1
    float_bits = (bits >> np.uint32(9)) | np.uint32(0x3F800000)
    floats = float_bits.view(np.float32) - np.float32(1.0)
    return floats < np.float32(p)


def _const_masks():
    b1, b2 = _threefry2x32(np.uint32(0), np.uint32(42),
                           np.zeros(3, np.uint32),
                           np.arange(3, dtype=np.uint32))
    dk = np.stack([b1, b2], axis=1)
    keep = 1.0 - RATE
    m0 = _np_bernoulli(dk[0], keep, ND * D).reshape(ND, D)
    m1 = _np_bernoulli(dk[1], keep, ND * H).reshape(ND, H)
    m2 = _np_bernoulli(dk[2], keep, ND)
    m1p = np.zeros((ND, D), dtype=bool)
    m1p[:, :H] = m1
    m01 = np.ascontiguousarray(m0.T).astype(np.int8)
    m01 += 2 * np.ascontiguousarray(m1p.T).astype(np.int8)
    s2m = np.where(m2, np.float32(INV_KEEP), np.float32(0.0))
    return m01, s2m


_M01T, _S2M = _const_masks()

NCHUNK = 5
CSZ = ND // NCHUNK
NREP = 4
_GOFF = ((((np.arange(ND) % CSZ) // (CSZ // 32)) % NREP)
         * (V + 1)).astype(np.int32)

NROW = 1024
NFLAT = NROW * B
NACC = 1008 * B
_ZEROS = np.zeros((NACC,), np.float32)
NC = 2
NS = 16
NW = NC * NS
TB = 2048


def _sc_gather(emb, bow):
    n = bow.shape[0]
    KT = n // NW
    KJ = KT // 128
    mesh = plsc.VectorSubcoreMesh(core_axis_name="c", subcore_axis_name="s")

    nbuf = 3

    @functools.partial(
        pl.kernel, mesh=mesh,
        out_type=jax.ShapeDtypeStruct((n, D), jnp.float32),
        scratch_types=[
            pltpu.VMEM((KT,), jnp.int32),
            *([pltpu.VMEM((128, D), jnp.float32)] * nbuf),
            *([pltpu.SemaphoreType.DMA] * nbuf),
        ],
    )
    def k(emb_hbm, bow_hbm, e_hbm, idx_v, *bufsem):
        bufs, sems = bufsem[:nbuf], bufsem[nbuf:]
        w = lax.axis_index("s") * NC + lax.axis_index("c")
        pltpu.sync_copy(bow_hbm.at[pl.ds(w * KT, KT)], idx_v)
        cps = [None] * nbuf
        for j in range(min(nbuf - 1, KJ)):
            cps[j] = pltpu.async_copy(
                emb_hbm.at[idx_v.at[pl.ds(j * 128, 128)]], bufs[j], sems[j])
        for j in range(KJ):
            nx = j + nbuf - 1
            if nx < KJ:
                cps[nx % nbuf] = pltpu.async_copy(
                    emb_hbm.at[idx_v.at[pl.ds(nx * 128, 128)]],
                    bufs[nx % nbuf], sems[nx % nbuf])
            cps[j % nbuf].wait()
            pltpu.sync_copy(bufs[j % nbuf],
                            e_hbm.at[pl.ds(w * KT + j * 128, 128)])

    return k(emb, bow)


def _sc_scatter(idx2, val2):
    R = idx2.shape[0]
    K = R // NW
    SL = NACC // NS // 128 * 128
    SLT = NACC - NS * SL
    SR = NROW // NS
    mesh = plsc.VectorSubcoreMesh(core_axis_name="c", subcore_axis_name="s")
    zeros = jnp.asarray(_ZEROS)

    @functools.partial(
        pl.kernel, mesh=mesh,
        out_type=jax.ShapeDtypeStruct((NC, NROW, B), jnp.float32),
        scratch_types=[
            pltpu.VMEM((K, 128), jnp.int32),
            pltpu.VMEM((K, 128), jnp.float32),
            pltpu.VMEM((8, B), jnp.float32),
            pltpu.VMEM_SHARED((NACC,), jnp.float32),
        ],
    )
    def k(zero_hbm, idx_hbm, val_hbm, out_hbm, idx_v, val_v, bounce, acc_sh):
        c = lax.axis_index("c")
        s = lax.axis_index("s")
        w = s * NC + c
        pltpu.sync_copy(zero_hbm.at[pl.ds(s * SL, SL)],
                        acc_sh.at[pl.ds(s * SL, SL)])
        if SLT:
            @pl.when(s == NS - 1)
            def _():
                pltpu.sync_copy(zero_hbm.at[pl.ds(NS * SL, SLT)],
                                acc_sh.at[pl.ds(NS * SL, SLT)])

        pltpu.sync_copy(idx_hbm.at[pl.ds(w * K, K)], idx_v)
        pltpu.sync_copy(val_hbm.at[pl.ds(w * K, K)], val_v)
        plsc.subcore_barrier()

        def body(j, carry):
            pltpu.sync_copy(val_v.at[j], acc_sh.at[idx_v.at[j]], add=True)
            return carry

        lax.fori_loop(0, K, body, 0)
        plsc.subcore_barrier()

        def dump(g, carry):
            def dump_row(r, carry2):
                row = s * SR + g * 8 + r

                @pl.when(row < NACC // B)
                def _():
                    pltpu.sync_copy(acc_sh.at[pl.ds(row * B, B)],
                                    bounce.at[r])

                @pl.when(row >= NACC // B)
                def _():
                    pltpu.sync_copy(zero_hbm.at[pl.ds(0, B)], bounce.at[r])

                return carry2

            lax.fori_loop(0, 8, dump_row, 0)
            pltpu.sync_copy(bounce, out_hbm.at[c, pl.ds(s * SR + g * 8, 8)])
            return carry

        lax.fori_loop(0, SR // 8, dump, 0)

    return k(zeros, idx2, val2)


def _pad_updates(flat_idx, vals):
    n = flat_idx.shape[0]
    kp = -(-(n // 128 // NW) // 8) * 8
    rp = NW * kp
    pad = rp * 128 - n
    if pad:
        pad_idx = (jnp.arange(pad, dtype=jnp.int32) * 64) % NACC
        flat_idx = jnp.concatenate([flat_idx, pad_idx])
        vals = jnp.concatenate([vals, jnp.zeros((pad,), jnp.float32)])
    return flat_idx.reshape(rp, 128), vals.reshape(rp, 128)


def _tc_tdv(e, m01t, w1t, b1c, w28, b2v, s2m, freqr, off):
    def body(e_ref, m_ref, w1_ref, b1_ref, w2_ref, b2_ref,
             s2_ref, fr_ref, out_ref):
        m = m_ref[...]
        xt = e_ref[...].T
        xm = jnp.where((m & 1) != 0, xt * INV_KEEP, 0.0)
        h = jnp.dot(w1_ref[...], xm, preferred_element_type=jnp.float32)
        h = jnp.maximum(h + b1_ref[...], 0.0)
        hm = jnp.where((m & 2) != 0, h * INV_KEEP, 0.0)
        s8 = jnp.dot(w2_ref[...], hm, preferred_element_type=jnp.float32)
        s = s8[0:1, :] + b2_ref[0, 0]
        out_ref[...] = (jnp.maximum(s, 0.0) * s2_ref[0]
                        * fr_ref[0]).reshape(1, 1, TB)

    nblk = e.shape[0] // TB
    return pl.pallas_call(
        body,
        grid=(nblk,),
        in_specs=[
            pl.BlockSpec((TB, D), lambda i: (i, 0)),
            pl.BlockSpec((D, TB), lambda i: (0, i + off)),
            pl.BlockSpec((D, D), lambda i: (0, 0)),
            pl.BlockSpec((D, 1), lambda i: (0, 0)),
            pl.BlockSpec((8, D), lambda i: (0, 0)),
            pl.BlockSpec((1, 1), lambda i: (0, 0)),
            pl.BlockSpec((1, 1, TB), lambda i: (i + off, 0, 0)),
            pl.BlockSpec((1, 1, TB), lambda i: (i + off, 0, 0)),
        ],
        out_specs=pl.BlockSpec((1, 1, TB), lambda i: (i, 0, 0)),
        out_shape=jax.ShapeDtypeStruct((nblk, 1, TB), jnp.float32),
    )(e, m01t, w1t, b1c, w28, b2v, s2m, freqr)


def _tc_bm25(qp, dp, k1v, bv):
    def body(qp_ref, dp_ref, k1_ref, b_ref, d_ref, rel_ref):
        d = dp_ref[0] + dp_ref[1]
        q = qp_ref[0] + qp_ref[1]
        row_sum = jnp.sum(d, axis=1, keepdims=True)
        maxdf = jnp.max(row_sum)
        idf = jnp.log((maxdf + 1.0) / (1.0 + row_sum))
        d_len = jnp.sum(d, axis=0, keepdims=True)
        avg = jnp.sum(d_len) / B
        k1 = k1_ref[0, 0]
        b = b_ref[0, 0]
        denom = d + k1 * (1.0 - b + b * (d_len / avg))
        bm = idf * ((k1 + 1.0) * d) / denom
        d_ref[...] = d[:V + 1]
        rel_ref[...] = jnp.sum(q * bm, axis=0)

    return pl.pallas_call(
        body,
        in_specs=[
            pl.BlockSpec((NC, NROW, B), lambda: (0, 0, 0)),
            pl.BlockSpec((NC, NROW, B), lambda: (0, 0, 0)),
            pl.BlockSpec((1, 1), lambda: (0, 0)),
            pl.BlockSpec((1, 1), lambda: (0, 0)),
        ],
        out_specs=[
            pl.BlockSpec((V + 1, B), lambda: (0, 0)),
            pl.BlockSpec((B,), lambda: (0,)),
        ],
        out_shape=[
            jax.ShapeDtypeStruct((V + 1, B), jnp.float32),
            jax.ShapeDtypeStruct((B,), jnp.float32),
        ],
    )(qp, dp, k1v, bv)


def kernel(q_indices_sparse_tensor_batch, q_frequencies_bow_batch,
           d_indices_sparse_tensor_batch, d_indices_bow_batch,
           d_frequencies_bow_batch, batch_size,
           emb, W1, b1, W2, b2, k1, b):
    del batch_size

    m01t = jnp.asarray(_M01T)
    s2m = jnp.asarray(_S2M.reshape(ND // TB, 1, TB))
    freqr = d_frequencies_bow_batch.reshape(ND // TB, 1, TB)

    w1t = jnp.pad(W1, ((0, 0), (0, D - H))).T
    b1c = jnp.pad(b1, (0, D - H)).reshape(D, 1)
    w28 = jnp.concatenate(
        [jnp.pad(W2[:, 0], (0, D - H)).reshape(1, D),
         jnp.zeros((7, D), jnp.float32)])
    b2v = b2.reshape(1, 1)
    k1v = jnp.float32(k1).reshape(1, 1)
    bv = jnp.float32(b).reshape(1, 1)

    qi, qv = _pad_updates(
        q_indices_sparse_tensor_batch[:, 0] * B
        + q_indices_sparse_tensor_batch[:, 1],
        q_frequencies_bow_batch)
    di_flat = (d_indices_sparse_tensor_batch[:, 0] * B
               + d_indices_sparse_tensor_batch[:, 1])
    bow = d_indices_bow_batch.astype(jnp.int32)

    if NREP > 1:
        emb_rep = jnp.concatenate([emb] * NREP, axis=0)
        bow_off = bow + jnp.asarray(_GOFF)
    else:
        emb_rep, bow_off = emb, bow
    tdv_parts = []
    for ci in range(NCHUNK):
        if ci == 0:
            table, idxs = emb, lax.slice(bow, (0,), (CSZ,))
        else:
            table = emb_rep
            idxs = lax.slice(bow_off, (ci * CSZ,), ((ci + 1) * CSZ,))
        e_c = _sc_gather(table, idxs)
        tdv_parts.append(_tc_tdv(e_c, m01t, w1t, b1c, w28, b2v, s2m, freqr,
                                 ci * (CSZ // TB)))
    qp = _sc_scatter(qi, qv)
    freq_tdv = jnp.concatenate(tdv_parts).reshape(ND)
    di, dv = _pad_updates(di_flat, freq_tdv)
    dp = _sc_scatter(di, dv)

    d_final, rel = _tc_bm25(qp, dp, k1v, bv)
    return (rel, d_final)


# speedup vs baseline: 1.0134x; 1.0134x over previous
import functools

import jax
import jax.numpy as jnp
import numpy as np
from jax import lax
from jax.experimental import pallas as pl
from jax.experimental.pallas import tpu as pltpu
from jax.experimental.pallas import tpu_sc as plsc

V = 1000
D = 128
B = 1024
ND = 204800
NQ = 20480
H = 100
RATE = 0.1
INV_KEEP = 1.0 / (1.0 - RATE)

def _threefry2x32(ks0, ks1, x0, x1):
    u32 = np.uint32
    ks2 = u32(ks0 ^ ks1 ^ u32(0x1BD11BDA))
    keys = (u32(ks0), u32(ks1), ks2)
    x0 = (x0 + keys[0]).astype(u32)
    x1 = (x1 + keys[1]).astype(u32)
    rot_a = (13, 15, 26, 6)
    rot_b = (17, 29, 16, 24)
    for i, rots in enumerate((rot_a, rot_b, rot_a, rot_b, rot_a)):
        for r in rots:
            x0 = (x0 + x1).astype(u32)
            x1 = ((x1 << u32(r)) | (x1 >> u32(32 - r))).astype(u32)
            x1 = (x1 ^ x0).astype(u32)
        x0 = (x0 + keys[(i + 1) % 3]).astype(u32)
        x1 = (x1 + keys[(i + 2) % 3] + u32(i + 1)).astype(u32)
    return x0, x1


def _np_bernoulli(key, p, n):
    o0, o1 = _threefry2x32(key[0], key[1], np.zeros(n, np.uint32),
                           np.arange(n, dtype=np.uint32))
    bits = o0 ^ o1
    float_bits = (bits >> np.uint32(9)) | np.uint32(0x3F800000)
    floats = float_bits.view(np.float32) - np.float32(1.0)
    return floats < np.float32(p)


def _const_masks():
    b1, b2 = _threefry2x32(np.uint32(0), np.uint32(42),
                           np.zeros(3, np.uint32),
                           np.arange(3, dtype=np.uint32))
    dk = np.stack([b1, b2], axis=1)
    keep = 1.0 - RATE
    m0 = _np_bernoulli(dk[0], keep, ND * D).reshape(ND, D)
    m1 = _np_bernoulli(dk[1], keep, ND * H).reshape(ND, H)
    m2 = _np_bernoulli(dk[2], keep, ND)
    m1p = np.zeros((ND, D), dtype=bool)
    m1p[:, :H] = m1
    m01 = np.ascontiguousarray(m0.T).astype(np.int8)
    m01 += 2 * np.ascontiguousarray(m1p.T).astype(np.int8)
    s2m = np.where(m2, np.float32(INV_KEEP), np.float32(0.0))
    return m01, s2m


_M01T, _S2M = _const_masks()

NCHUNK = 5
CSZ = ND // NCHUNK
NREP = 4
_GOFF = ((((np.arange(ND) % CSZ) // (CSZ // 32)) % NREP)
         * (V + 1)).astype(np.int32)

NROW = 1024
NFLAT = NROW * B
NACC = 1008 * B
_ZEROS = np.zeros((NACC,), np.float32)
NC = 2
NS = 16
NW = NC * NS
TB = 2048


def _sc_gather(emb, bow):
    n = bow.shape[0]
    KT = n // NW
    KJ = KT // 128
    mesh = plsc.VectorSubcoreMesh(core_axis_name="c", subcore_axis_name="s")

    nbuf = 3

    @functools.partial(
        pl.kernel, mesh=mesh,
        out_type=jax.ShapeDtypeStruct((n, D), jnp.float32),
        scratch_types=[
            pltpu.VMEM((KT,), jnp.int32),
            *([pltpu.VMEM((128, D), jnp.float32)] * nbuf),
            *([pltpu.SemaphoreType.DMA] * nbuf),
        ],
    )
    def k(emb_hbm, bow_hbm, e_hbm, idx_v, *bufsem):
        bufs, sems = bufsem[:nbuf], bufsem[nbuf:]
        w = lax.axis_index("s") * NC + lax.axis_index("c")
        pltpu.sync_copy(bow_hbm.at[pl.ds(w * KT, KT)], idx_v)
        cps = [None] * nbuf
        for j in range(min(nbuf - 1, KJ)):
            cps[j] = pltpu.async_copy(
                emb_hbm.at[idx_v.at[pl.ds(j * 128, 128)]], bufs[j], sems[j])
        for j in range(KJ):
            nx = j + nbuf - 1
            if nx < KJ:
                cps[nx % nbuf] = pltpu.async_copy(
                    emb_hbm.at[idx_v.at[pl.ds(nx * 128, 128)]],
                    bufs[nx % nbuf], sems[nx % nbuf])
            cps[j % nbuf].wait()
            pltpu.sync_copy(bufs[j % nbuf],
                            e_hbm.at[pl.ds(w * KT + j * 128, 128)])

    return k(emb, bow)


def _sc_scatter(idx2, val2):
    R = idx2.shape[0]
    K = R // NW
    SL = NACC // NS // 128 * 128
    SLT = NACC - NS * SL
    SR = NROW // NS
    mesh = plsc.VectorSubcoreMesh(core_axis_name="c", subcore_axis_name="s")
    zeros = jnp.asarray(_ZEROS)

    @functools.partial(
        pl.kernel, mesh=mesh,
        out_type=jax.ShapeDtypeStruct((NC, NROW, B), jnp.float32),
        scratch_types=[
            pltpu.VMEM((K, 128), jnp.int32),
            pltpu.VMEM((K, 128), jnp.float32),
            pltpu.VMEM((8, B), jnp.float32),
            pltpu.VMEM_SHARED((NACC,), jnp.float32),
        ],
    )
    def k(zero_hbm, idx_hbm, val_hbm, out_hbm, idx_v, val_v, bounce, acc_sh):
        c = lax.axis_index("c")
        s = lax.axis_index("s")
        w = s * NC + c
        pltpu.sync_copy(zero_hbm.at[pl.ds(s * SL, SL)],
                        acc_sh.at[pl.ds(s * SL, SL)])
        if SLT:
            @pl.when(s == NS - 1)
            def _():
                pltpu.sync_copy(zero_hbm.at[pl.ds(NS * SL, SLT)],
                                acc_sh.at[pl.ds(NS * SL, SLT)])

        pltpu.sync_copy(idx_hbm.at[pl.ds(w * K, K)], idx_v)
        pltpu.sync_copy(val_hbm.at[pl.ds(w * K, K)], val_v)
        plsc.subcore_barrier()

        def body(j, carry):
            pltpu.sync_copy(val_v.at[j], acc_sh.at[idx_v.at[j]], add=True)
            return carry

        lax.fori_loop(0, K, body, 0)
        plsc.subcore_barrier()

        def dump(g, carry):
            def dump_row(r, carry2):
                row = s * SR + g * 8 + r

                @pl.when(row < NACC // B)
                def _():
                    pltpu.sync_copy(acc_sh.at[pl.ds(row * B, B)],
                                    bounce.at[r])

                @pl.when(row >= NACC // B)
                def _():
                    pltpu.sync_copy(zero_hbm.at[pl.ds(0, B)], bounce.at[r])

                return carry2

            lax.fori_loop(0, 8, dump_row, 0)
            pltpu.sync_copy(bounce, out_hbm.at[c, pl.ds(s * SR + g * 8, 8)])
            return carry

        lax.fori_loop(0, SR // 8, dump, 0)

    return k(zeros, idx2, val2)


def _pad_updates(flat_idx, vals):
    n = flat_idx.shape[0]
    kp = -(-(n // 128 // NW) // 8) * 8
    rp = NW * kp
    pad = rp * 128 - n
    if pad:
        pad_idx = (jnp.arange(pad, dtype=jnp.int32) * 64) % NACC
        flat_idx = jnp.concatenate([flat_idx, pad_idx])
        vals = jnp.concatenate([vals, jnp.zeros((pad,), jnp.float32)])
    return flat_idx.reshape(rp, 128), vals.reshape(rp, 128)


def _tc_tdv(e, m01t, w1t, b1c, w28, b2v, s2m, freqr, off):
    def body(e_ref, m_ref, w1_ref, b1_ref, w2_ref, b2_ref,
             s2_ref, fr_ref, out_ref):
        m = m_ref[...]
        xt = e_ref[...].T
        xm = jnp.where((m & 1) != 0, xt * INV_KEEP, 0.0)
        h = jnp.dot(w1_ref[...], xm, preferred_element_type=jnp.float32)
        h = jnp.maximum(h + b1_ref[...], 0.0)
        hm = jnp.where((m & 2) != 0, h * INV_KEEP, 0.0)
        s8 = jnp.dot(w2_ref[...], hm, preferred_element_type=jnp.float32)
        s = s8[0:1, :] + b2_ref[0, 0]
        out_ref[...] = (jnp.maximum(s, 0.0) * s2_ref[0]
                        * fr_ref[0]).reshape(1, 1, TB)

    nblk = e.shape[0] // TB
    return pl.pallas_call(
        body,
        grid=(nblk,),
        in_specs=[
            pl.BlockSpec((TB, D), lambda i: (i, 0)),
            pl.BlockSpec((D, TB), lambda i: (0, i + off)),
            pl.BlockSpec((D, D), lambda i: (0, 0)),
            pl.BlockSpec((D, 1), lambda i: (0, 0)),
            pl.BlockSpec((8, D), lambda i: (0, 0)),
            pl.BlockSpec((1, 1), lambda i: (0, 0)),
            pl.BlockSpec((1, 1, TB), lambda i: (i + off, 0, 0)),
            pl.BlockSpec((1, 1, TB), lambda i: (i + off, 0, 0)),
        ],
        out_specs=pl.BlockSpec((1, 1, TB), lambda i: (i, 0, 0)),
        out_shape=jax.ShapeDtypeStruct((nblk, 1, TB), jnp.float32),
    )(e, m01t, w1t, b1c, w28, b2v, s2m, freqr)


def _tc_bm25(qp, dp, k1v, bv):
    def body(qp_ref, dp_ref, k1_ref, b_ref, d_ref, rel_ref):
        d = dp_ref[0] + dp_ref[1]
        q = qp_ref[0] + qp_ref[1]
        row_sum = jnp.sum(d, axis=1, keepdims=True)
        maxdf = jnp.max(row_sum)
        idf = jnp.log((maxdf + 1.0) / (1.0 + row_sum))
        d_len = jnp.sum(d, axis=0, keepdims=True)
        avg = jnp.sum(d_len) / B
        k1 = k1_ref[0, 0]
        b = b_ref[0, 0]
        denom = d + k1 * (1.0 - b + b * (d_len / avg))
        bm = idf * ((k1 + 1.0) * d) / denom
        d_ref[...] = d[:V + 1]
        rel_ref[...] = jnp.sum(q * bm, axis=0)

    return pl.pallas_call(
        body,
        in_specs=[
            pl.BlockSpec((NC, NROW, B), lambda: (0, 0, 0)),
            pl.BlockSpec((NC, NROW, B), lambda: (0, 0, 0)),
            pl.BlockSpec((1, 1), lambda: (0, 0)),
            pl.BlockSpec((1, 1), lambda: (0, 0)),
        ],
        out_specs=[
            pl.BlockSpec((V + 1, B), lambda: (0, 0)),
            pl.BlockSpec((B,), lambda: (0,)),
        ],
        out_shape=[
            jax.ShapeDtypeStruct((V + 1, B), jnp.float32),
            jax.ShapeDtypeStruct((B,), jnp.float32),
        ],
    )(qp, dp, k1v, bv)


def kernel(q_indices_sparse_tensor_batch, q_frequencies_bow_batch,
           d_indices_sparse_tensor_batch, d_indices_bow_batch,
           d_frequencies_bow_batch, batch_size,
           emb, W1, b1, W2, b2, k1, b):
    del batch_size

    m01t = jnp.asarray(_M01T)
    s2m = jnp.asarray(_S2M.reshape(ND // TB, 1, TB))
    freqr = d_frequencies_bow_batch.reshape(ND // TB, 1, TB)

    w1t = jnp.pad(W1, ((0, 0), (0, D - H))).T
    b1c = jnp.pad(b1, (0, D - H)).reshape(D, 1)
    w28 = jnp.concatenate(
        [jnp.pad(W2[:, 0], (0, D - H)).reshape(1, D),
         jnp.zeros((7, D), jnp.float32)])
    b2v = b2.reshape(1, 1)
    k1v = jnp.float32(k1).reshape(1, 1)
    bv = jnp.float32(b).reshape(1, 1)

    qi, qv = _pad_updates(
        q_indices_sparse_tensor_batch[:, 0] * B
        + q_indices_sparse_tensor_batch[:, 1],
        q_frequencies_bow_batch)
    di_flat = (d_indices_sparse_tensor_batch[:, 0] * B
               + d_indices_sparse_tensor_batch[:, 1])
    bow = d_indices_bow_batch.astype(jnp.int32)

    if NREP > 1:
        emb_rep = jnp.concatenate([emb] * NREP, axis=0)
        bow_off = bow + jnp.asarray(_GOFF)
    else:
        emb_rep, bow_off = emb, bow
    tdv_parts = []
    for ci in range(NCHUNK):
        e_c = _sc_gather(emb_rep,
                         lax.slice(bow_off, (ci * CSZ,), ((ci + 1) * CSZ,)))
        tdv_parts.append(_tc_tdv(e_c, m01t, w1t, b1c, w28, b2v, s2m, freqr,
                                 ci * (CSZ // TB)))
    qp = _sc_scatter(qi, qv)
    freq_tdv = jnp.concatenate(tdv_parts).reshape(ND)
    di, dv = _pad_updates(di_flat, freq_tdv)
    dp = _sc_scatter(di, dv)

    d_final, rel = _tc_bm25(qp, dp, k1v, bv)
    return (rel, d_final)
